# baseline (speedup 1.0000x reference)
import functools

import jax
import jax.numpy as jnp
from jax import lax
from jax.experimental import pallas as pl
from jax.experimental.pallas import tpu as pltpu
from jax.experimental.pallas import tpu_sc as plsc

N, S, E = 4, 4096, 1024
NW = 32
SEQ_PER_W = S // NW
CS = 16
NCHUNK = SEQ_PER_W // CS
VPR = E // 16

_mesh = plsc.VectorSubcoreMesh(core_axis_name="c", subcore_axis_name="s")


@functools.partial(
    pl.kernel,
    mesh=_mesh,
    out_type=jax.ShapeDtypeStruct((N * S, E), jnp.float32),
    scratch_types=[
        pltpu.VMEM((CS, E), jnp.float32),
        pltpu.VMEM((N, CS, E), jnp.float32),
    ],
)
def _sc_add(x_hbm, t_hbm, out_hbm, t_buf, x_buf):
    wid = lax.axis_index("s") * 2 + lax.axis_index("c")
    seq0 = wid * SEQ_PER_W

    def chunk(c, carry):
        s0 = seq0 + c * CS
        pltpu.sync_copy(t_hbm.at[pl.ds(s0, CS)], t_buf)
        for n in range(N):
            pltpu.sync_copy(x_hbm.at[pl.ds(n * S + s0, CS)], x_buf.at[n])

        @plsc.parallel_loop(0, CS * VPR, unroll=8)
        def body(i):
            r = i // VPR
            j = (i % VPR) * 16
            t = t_buf[r, pl.ds(j, 16)]
            for n in range(N):
                x_buf[n, r, pl.ds(j, 16)] = x_buf[n, r, pl.ds(j, 16)] + t

        for n in range(N):
            pltpu.sync_copy(x_buf.at[n], out_hbm.at[pl.ds(n * S + s0, CS)])
        return carry

    lax.fori_loop(0, NCHUNK, chunk, 0)


def kernel(x, table):
    out = _sc_add(x.reshape(N * S, E), table)
    return out.reshape(N, S, E)


# speedup vs baseline: 1.5475x; 1.5475x over previous
import functools

import jax
import jax.numpy as jnp
from jax import lax
from jax.experimental import pallas as pl
from jax.experimental.pallas import tpu as pltpu
from jax.experimental.pallas import tpu_sc as plsc

N, S, E = 4, 4096, 1024
NW = 32
SEQ_PER_W = S // NW
CS = 8
NCHUNK = SEQ_PER_W // CS
VPR = E // 16

_mesh = plsc.VectorSubcoreMesh(core_axis_name="c", subcore_axis_name="s")


@functools.partial(
    pl.kernel,
    mesh=_mesh,
    out_type=jax.ShapeDtypeStruct((N, S, E), jnp.float32),
    scratch_types=[
        pltpu.VMEM((2, CS, E), jnp.float32),
        pltpu.VMEM((3, N, CS, E), jnp.float32),
        [pltpu.SemaphoreType.DMA] * 2,
        [pltpu.SemaphoreType.DMA] * 3,
        [pltpu.SemaphoreType.DMA] * 3,
    ],
)
def _sc_add(x_hbm, t_hbm, out_hbm, t_buf, x_buf, sem_t, sem_x, sem_o):
    wid = lax.axis_index("s") * 2 + lax.axis_index("c")
    seq0 = wid * SEQ_PER_W

    def start_in(c):
        s0 = seq0 + c * CS
        dt = pltpu.async_copy(t_hbm.at[pl.ds(s0, CS)], t_buf.at[c % 2],
                              sem_t[c % 2])
        dx = pltpu.async_copy(x_hbm.at[:, pl.ds(s0, CS)], x_buf.at[c % 3],
                              sem_x[c % 3])
        return dt, dx

    descs_in = [None, None, None]
    descs_out = [None, None, None]
    descs_in[0] = start_in(0)

    for c in range(NCHUNK):
        b = c % 3
        bt = c % 2
        for d in descs_in[b]:
            d.wait()
        if c + 1 < NCHUNK:
            bn = (c + 1) % 3
            if descs_out[bn] is not None:
                descs_out[bn].wait()
                descs_out[bn] = None
            descs_in[bn] = start_in(c + 1)

        @plsc.parallel_loop(0, CS * VPR, unroll=8)
        def body(i):
            r = i // VPR
            j = (i % VPR) * 16
            t = t_buf[bt, r, pl.ds(j, 16)]
            for n in range(N):
                x_buf[b, n, r, pl.ds(j, 16)] = x_buf[b, n, r, pl.ds(j, 16)] + t

        s0 = seq0 + c * CS
        descs_out[b] = pltpu.async_copy(x_buf.at[b],
                                        out_hbm.at[:, pl.ds(s0, CS)],
                                        sem_o[b])

    for d in descs_out:
        if d is not None:
            d.wait()


def kernel(x, table):
    return _sc_add(x, table)
